# Initial kernel scaffold; baseline (speedup 1.0000x reference)
import functools

import jax
import jax.numpy as jnp
from jax import lax
from jax.experimental import pallas as pl
from jax.experimental.pallas import tpu as pltpu
from jax.experimental.pallas import tpu_sc as plsc

NN = 10000
EE = 320000
DD = 128
NPAD = 10240
NC = 2
NS = 16
NW = NC * NS
EPW = EE // NW
CH = 125
NCH = EPW // CH
RPT = NPAD // NS
DEGW = 16


def _mesh():
    return plsc.VectorSubcoreMesh(core_axis_name="c", subcore_axis_name="s")


@functools.partial(
    pl.kernel,
    mesh=_mesh(),
    out_type=jax.ShapeDtypeStruct((NC * NPAD, DD), jnp.float32),
    scratch_types=[
        pltpu.VMEM((NCH, CH), jnp.int32),
        pltpu.VMEM((NCH, CH), jnp.int32),
        pltpu.VMEM((CH, DD), jnp.float32),
        pltpu.VMEM((64, DD), jnp.float32),
        pltpu.VMEM_SHARED((NPAD, DD), jnp.float32),
    ],
)
def _sc_agg(h_hbm, src_hbm, dst_hbm, zeros_hbm, out_hbm,
            src_v, dst_v, stage_v, zero_v, acc_sh):
    c = lax.axis_index("c")
    s = lax.axis_index("s")
    wid = c * NS + s
    base = s * RPT
    pltpu.sync_copy(zeros_hbm, zero_v)
    for k in range(RPT // 64):
        pltpu.sync_copy(zero_v, acc_sh.at[pl.ds(base + k * 64, 64)])
    pltpu.sync_copy(src_hbm.at[wid], src_v)
    pltpu.sync_copy(dst_hbm.at[wid], dst_v)
    plsc.subcore_barrier()

    def body(j, carry):
        pltpu.sync_copy(h_hbm.at[src_v.at[j]], stage_v)
        pltpu.sync_copy(stage_v, acc_sh.at[dst_v.at[j]], add=True)
        return carry

    lax.fori_loop(0, NCH, body, 0)
    plsc.subcore_barrier()
    pltpu.sync_copy(acc_sh.at[pl.ds(base, RPT)],
                    out_hbm.at[pl.ds(c * NPAD + base, RPT)])


@functools.partial(
    pl.kernel,
    mesh=_mesh(),
    out_type=jax.ShapeDtypeStruct((NC * NPAD, DEGW), jnp.float32),
    scratch_types=[
        pltpu.VMEM((NCH, CH), jnp.int32),
        pltpu.VMEM((CH, DEGW), jnp.float32),
        pltpu.VMEM_SHARED((NPAD, DEGW), jnp.float32),
    ],
)
def _sc_deg(dst_hbm, ones_hbm, zeros_hbm, out_hbm, dst_v, ones_v, acc_sh):
    c = lax.axis_index("c")
    s = lax.axis_index("s")
    wid = c * NS + s
    base = s * RPT
    pltpu.sync_copy(zeros_hbm, acc_sh.at[pl.ds(base, RPT)])
    pltpu.sync_copy(dst_hbm.at[wid], dst_v)
    pltpu.sync_copy(ones_hbm, ones_v)
    plsc.subcore_barrier()

    def body(j, carry):
        pltpu.sync_copy(ones_v, acc_sh.at[dst_v.at[j]], add=True)
        return carry

    lax.fori_loop(0, NCH, body, 0)
    plsc.subcore_barrier()
    pltpu.sync_copy(acc_sh.at[pl.ds(base, RPT)],
                    out_hbm.at[pl.ds(c * NPAD + base, RPT)])


def _tc_body(relu):
    def f(p0, p1, d0, d1, h, wl, blp, wr, o):
        deg = jnp.maximum(d0[...][:, :1] + d1[...][:, :1], 1.0)
        agg = (p0[...] + p1[...]) / deg
        y = lax.dot_general(agg, wl[...], (((1,), (1,)), ((), ())),
                            preferred_element_type=jnp.float32)
        y = y + blp[...][0:1, :]
        y = y + lax.dot_general(h[...], wr[...], (((1,), (1,)), ((), ())),
                                preferred_element_type=jnp.float32)
        if relu:
            y = jnp.maximum(y, 0.0)
        o[...] = y
    return f


_BM = 1024
_NBLK = NPAD // _BM


def _dense(parts, deg, h, wl, bl, wr, relu):
    blp = jnp.broadcast_to(bl[None, :], (8, DD))
    return pl.pallas_call(
        _tc_body(relu),
        grid=(_NBLK,),
        in_specs=[
            pl.BlockSpec((_BM, DD), lambda i: (i, 0)),
            pl.BlockSpec((_BM, DD), lambda i: (_NBLK + i, 0)),
            pl.BlockSpec((_BM, DEGW), lambda i: (i, 0)),
            pl.BlockSpec((_BM, DEGW), lambda i: (_NBLK + i, 0)),
            pl.BlockSpec((_BM, DD), lambda i: (i, 0)),
            pl.BlockSpec((DD, DD), lambda i: (0, 0)),
            pl.BlockSpec((8, DD), lambda i: (0, 0)),
            pl.BlockSpec((DD, DD), lambda i: (0, 0)),
        ],
        out_specs=pl.BlockSpec((_BM, DD), lambda i: (i, 0)),
        out_shape=jax.ShapeDtypeStruct((NPAD, DD), jnp.float32),
    )(parts, parts, deg, deg, h, wl, blp, wr)


def kernel(x, edge_index, Wl1, bl1, Wr1, Wl2, bl2, Wr2, Wl3, bl3, Wr3):
    src = edge_index[0].reshape(NW, NCH, CH)
    dst = edge_index[1].reshape(NW, NCH, CH)
    xp = jnp.pad(x, ((0, NPAD - NN), (0, 0)))
    zeros_d = jnp.zeros((64, DD), jnp.float32)
    zeros_g = jnp.zeros((RPT, DEGW), jnp.float32)
    ones_g = jnp.ones((CH, DEGW), jnp.float32)

    deg = _sc_deg(dst, ones_g, zeros_g)
    h = xp
    for wl, bl, wr, relu in ((Wl1, bl1, Wr1, True),
                             (Wl2, bl2, Wr2, False),
                             (Wl3, bl3, Wr3, False)):
        parts = _sc_agg(h, src, dst, zeros_d)
        h = _dense(parts, deg, h, wl, bl, wr, relu)
    return h[:NN]


# speedup vs baseline: 10.8620x; 10.8620x over previous
import functools

import jax
import jax.numpy as jnp
from jax import lax
from jax.experimental import pallas as pl
from jax.experimental.pallas import tpu as pltpu
from jax.experimental.pallas import tpu_sc as plsc

NN = 10000
EE = 320000
DD = 128
NPAD = 10240
NC = 2
NS = 16
NW = NC * NS
EPW = EE // NW
CH = 80
NCH = EPW // CH
RPT = NPAD // NS


def _mesh():
    return plsc.VectorSubcoreMesh(core_axis_name="c", subcore_axis_name="s",
                                  num_cores=NC, num_subcores=NS)


def _sc_agg_body(h_hbm, src_hbm, dst_hbm, zeros_hbm, out_hbm,
                 src_v, dst_v, stage_v, zero_v, acc_sh, gsem0, gsem1):
    c = lax.axis_index("c")
    s = lax.axis_index("s")
    wid = c * NS + s
    base = s * RPT
    pltpu.sync_copy(zeros_hbm, zero_v)
    for k in range(RPT // 16):
        pltpu.sync_copy(zero_v, acc_sh.at[pl.ds(base + k * 16, 16)])
    pltpu.sync_copy(src_hbm.at[wid], src_v)
    pltpu.sync_copy(dst_hbm.at[wid], dst_v)
    plsc.subcore_barrier()
    gsems = (gsem0, gsem1)

    def start_gather(j, b):
        pltpu.async_copy(h_hbm.at[src_v.at[pl.ds(j * CH, CH)]],
                         stage_v.at[b], gsems[b])

    def wait_gather(j, b):
        pltpu.make_async_copy(h_hbm.at[src_v.at[pl.ds(j * CH, CH)]],
                              stage_v.at[b], gsems[b]).wait()

    start_gather(0, 0)

    def body(g, carry):
        for b in (0, 1):
            j = 2 * g + b
            start_gather(j + 1, 1 - b)
            wait_gather(j, b)
            pltpu.sync_copy(stage_v.at[b], acc_sh.at[dst_v.at[j]], add=True)
        return carry

    lax.fori_loop(0, (NCH - 1) // 2, body, 0)
    wait_gather(NCH - 1, 0)
    pltpu.sync_copy(stage_v.at[0], acc_sh.at[dst_v.at[NCH - 1]], add=True)
    plsc.subcore_barrier()
    pltpu.sync_copy(acc_sh.at[pl.ds(base, RPT)],
                    out_hbm.at[pl.ds(c * NPAD + base, RPT)])


def _sc_deg_body(dst_hbm, zeros_hbm, out_hbm, dst_v, acc_v):
    c = lax.axis_index("c")
    s = lax.axis_index("s")
    wid = c * NS + s
    pltpu.sync_copy(zeros_hbm, acc_v)
    pltpu.sync_copy(dst_hbm.at[wid], dst_v)
    ones = jnp.full((16,), 1.0, jnp.float32)

    def body(i, carry):
        idx = dst_v[pl.ds(i * 16, 16)]
        plsc.addupdate_scatter(acc_v, [idx], ones)
        return carry

    lax.fori_loop(0, EPW // 16, body, 0)
    pltpu.sync_copy(acc_v, out_hbm.at[wid])


@functools.lru_cache(maxsize=None)
def _sc_kernels():
    agg = pl.kernel(
        _sc_agg_body,
        mesh=_mesh(),
        out_type=jax.ShapeDtypeStruct((NC * NPAD, DD), jnp.float32),
        scratch_types=[
            pltpu.VMEM((EPW,), jnp.int32),
            pltpu.VMEM((NCH, CH), jnp.int32),
            pltpu.VMEM((2, CH, DD), jnp.float32),
            pltpu.VMEM((16, DD), jnp.float32),
            pltpu.VMEM_SHARED((NPAD, DD), jnp.float32),
            pltpu.SemaphoreType.DMA,
            pltpu.SemaphoreType.DMA,
        ],
    )
    deg = pl.kernel(
        _sc_deg_body,
        mesh=_mesh(),
        compiler_params=pltpu.CompilerParams(needs_layout_passes=False),
        out_type=jax.ShapeDtypeStruct((NW, NPAD), jnp.float32),
        scratch_types=[
            pltpu.VMEM((EPW,), jnp.int32),
            pltpu.VMEM((NPAD,), jnp.float32),
        ],
    )
    return agg, deg


def _tc_body(relu):
    def f(p0, p1, d, h, wl, blp, wr, o):
        deg = jnp.maximum(jnp.sum(d[...], axis=0), 1.0)[:, None]
        agg = (p0[...] + p1[...]) / deg
        y = lax.dot_general(agg, wl[...], (((1,), (1,)), ((), ())),
                            preferred_element_type=jnp.float32)
        y = y + blp[...][0:1, :]
        y = y + lax.dot_general(h[...], wr[...], (((1,), (1,)), ((), ())),
                                preferred_element_type=jnp.float32)
        if relu:
            y = jnp.maximum(y, 0.0)
        o[...] = y
    return f


_BM = 1024
_NBLK = NPAD // _BM


def _dense(parts, deg, h, wl, bl, wr, relu):
    blp = jnp.broadcast_to(bl[None, :], (8, DD))
    return pl.pallas_call(
        _tc_body(relu),
        grid=(_NBLK,),
        in_specs=[
            pl.BlockSpec((_BM, DD), lambda i: (i, 0)),
            pl.BlockSpec((_BM, DD), lambda i: (_NBLK + i, 0)),
            pl.BlockSpec((NW, _BM), lambda i: (0, i)),
            pl.BlockSpec((_BM, DD), lambda i: (i, 0)),
            pl.BlockSpec((DD, DD), lambda i: (0, 0)),
            pl.BlockSpec((8, DD), lambda i: (0, 0)),
            pl.BlockSpec((DD, DD), lambda i: (0, 0)),
        ],
        out_specs=pl.BlockSpec((_BM, DD), lambda i: (i, 0)),
        out_shape=jax.ShapeDtypeStruct((NPAD, DD), jnp.float32),
    )(parts, parts, deg, h, wl, blp, wr)


def kernel(x, edge_index, Wl1, bl1, Wr1, Wl2, bl2, Wr2, Wl3, bl3, Wr3):
    src = edge_index[0].reshape(NW, EPW)
    dst = edge_index[1].reshape(NW, NCH, CH)
    dst_flat = edge_index[1].reshape(NW, EPW)
    xp = jnp.pad(x, ((0, NPAD - NN), (0, 0)))
    zeros_d = jnp.zeros((16, DD), jnp.float32)
    zeros_1 = jnp.zeros((NPAD,), jnp.float32)

    sc_agg, sc_deg = _sc_kernels()
    deg = sc_deg(dst_flat, zeros_1)
    h = xp
    for wl, bl, wr, relu in ((Wl1, bl1, Wr1, True),
                             (Wl2, bl2, Wr2, False),
                             (Wl3, bl3, Wr3, False)):
        parts = sc_agg(h, src, dst, zeros_d)
        h = _dense(parts, deg, h, wl, bl, wr, relu)
    return h[:NN]


# speedup vs baseline: 12.8341x; 1.1816x over previous
import functools

import jax
import jax.numpy as jnp
from jax import lax
from jax.experimental import pallas as pl
from jax.experimental.pallas import tpu as pltpu
from jax.experimental.pallas import tpu_sc as plsc

NN = 10000
EE = 320000
DD = 128
NPAD = 10240
NC = 2
NS = 16
NW = NC * NS
EPW = EE // NW
CH = 80
NCH = EPW // CH
RPT = NPAD // NS


def _mesh():
    return plsc.VectorSubcoreMesh(core_axis_name="c", subcore_axis_name="s",
                                  num_cores=NC, num_subcores=NS)


_NBUF = 3
assert (NCH - 2) % _NBUF == 0


def _sc_agg_body(h_hbm, src_hbm, dst_hbm, zeros_hbm, out_hbm,
                 src_v, dstr_v, stage_v, zero_v, acc_sh,
                 g0, g1, g2, d0, d1, d2, s0, s1, s2):
    gsems = (g0, g1, g2)
    dsems = (d0, d1, d2)
    ssems = (s0, s1, s2)
    c = lax.axis_index("c")
    s = lax.axis_index("s")
    wid = c * NS + s
    base = s * RPT
    pltpu.sync_copy(zeros_hbm, zero_v)
    for k in range(RPT // 16):
        pltpu.sync_copy(zero_v, acc_sh.at[pl.ds(base + k * 16, 16)])
    pltpu.sync_copy(src_hbm.at[wid], src_v)
    plsc.subcore_barrier()

    def start_fetch(j, b):
        pltpu.async_copy(dst_hbm.at[wid, pl.ds(j, 1)],
                         dstr_v.at[pl.ds(b, 1)], dsems[b])
        pltpu.async_copy(h_hbm.at[src_v.at[pl.ds(j * CH, CH)]],
                         stage_v.at[b], gsems[b])

    def wait_fetch(j, b):
        pltpu.make_async_copy(dst_hbm.at[wid, pl.ds(j, 1)],
                              dstr_v.at[pl.ds(b, 1)], dsems[b]).wait()
        pltpu.make_async_copy(h_hbm.at[src_v.at[pl.ds(j * CH, CH)]],
                              stage_v.at[b], gsems[b]).wait()

    def start_scatter(b):
        pltpu.async_copy(stage_v.at[b], acc_sh.at[dstr_v.at[b]], ssems[b],
                         add=True)

    def wait_scatter(b):
        pltpu.make_async_copy(stage_v.at[b], acc_sh.at[dstr_v.at[b]],
                              ssems[b]).wait()

    start_fetch(0, 0)
    start_fetch(1, 1)

    def body(g, carry):
        for b in range(_NBUF):
            j = 3 * g + b
            bn = (b + 2) % _NBUF

            @pl.when(j >= 1)
            def _():
                wait_scatter(bn)
            start_fetch(j + 2, bn)
            wait_fetch(j, b)
            start_scatter(b)
        return carry

    ngrp = (NCH - 2) // _NBUF
    lax.fori_loop(0, ngrp, body, 0)
    for j in range(_NBUF * ngrp, NCH):
        b = j % _NBUF
        wait_scatter((b + 2) % _NBUF)
        wait_fetch(j, b)
        start_scatter(b)
    wait_scatter((NCH - 1) % _NBUF)
    plsc.subcore_barrier()
    pltpu.sync_copy(acc_sh.at[pl.ds(base, RPT)],
                    out_hbm.at[pl.ds(c * NPAD + base, RPT)])


def _sc_deg_body(dst_hbm, zeros_hbm, out_hbm, dst_v, acc_v):
    c = lax.axis_index("c")
    s = lax.axis_index("s")
    wid = c * NS + s
    pltpu.sync_copy(zeros_hbm, acc_v)
    pltpu.sync_copy(dst_hbm.at[wid], dst_v)
    ones = jnp.full((16,), 1.0, jnp.float32)

    def body(i, carry):
        idx = dst_v[pl.ds(i * 16, 16)]
        plsc.addupdate_scatter(acc_v, [idx], ones)
        return carry

    lax.fori_loop(0, EPW // 16, body, 0)
    pltpu.sync_copy(acc_v, out_hbm.at[wid])


@functools.lru_cache(maxsize=None)
def _sc_kernels():
    agg = pl.kernel(
        _sc_agg_body,
        mesh=_mesh(),
        out_type=jax.ShapeDtypeStruct((NC * NPAD, DD), jnp.float32),
        scratch_types=(
            [
                pltpu.VMEM((EPW,), jnp.int32),
                pltpu.VMEM((_NBUF, CH), jnp.int32),
                pltpu.VMEM((_NBUF, CH, DD), jnp.float32),
                pltpu.VMEM((16, DD), jnp.float32),
                pltpu.VMEM_SHARED((NPAD, DD), jnp.float32),
            ]
            + [pltpu.SemaphoreType.DMA] * 9
        ),
    )
    deg = pl.kernel(
        _sc_deg_body,
        mesh=_mesh(),
        compiler_params=pltpu.CompilerParams(needs_layout_passes=False),
        out_type=jax.ShapeDtypeStruct((NW, NPAD), jnp.float32),
        scratch_types=[
            pltpu.VMEM((EPW,), jnp.int32),
            pltpu.VMEM((NPAD,), jnp.float32),
        ],
    )
    return agg, deg


def _tc_body(relu):
    def f(p0, p1, d, h, wl, blp, wr, o):
        deg = jnp.maximum(jnp.sum(d[...], axis=0), 1.0)[:, None]
        agg = (p0[...] + p1[...]) / deg
        y = lax.dot_general(agg, wl[...], (((1,), (1,)), ((), ())),
                            preferred_element_type=jnp.float32)
        y = y + blp[...][0:1, :]
        y = y + lax.dot_general(h[...], wr[...], (((1,), (1,)), ((), ())),
                                preferred_element_type=jnp.float32)
        if relu:
            y = jnp.maximum(y, 0.0)
        o[...] = y
    return f


_BM = 1024
_NBLK = NPAD // _BM


def _dense(parts, deg, h, wl, bl, wr, relu):
    blp = jnp.broadcast_to(bl[None, :], (8, DD))
    return pl.pallas_call(
        _tc_body(relu),
        grid=(_NBLK,),
        in_specs=[
            pl.BlockSpec((_BM, DD), lambda i: (i, 0)),
            pl.BlockSpec((_BM, DD), lambda i: (_NBLK + i, 0)),
            pl.BlockSpec((NW, _BM), lambda i: (0, i)),
            pl.BlockSpec((_BM, DD), lambda i: (i, 0)),
            pl.BlockSpec((DD, DD), lambda i: (0, 0)),
            pl.BlockSpec((8, DD), lambda i: (0, 0)),
            pl.BlockSpec((DD, DD), lambda i: (0, 0)),
        ],
        out_specs=pl.BlockSpec((_BM, DD), lambda i: (i, 0)),
        out_shape=jax.ShapeDtypeStruct((NPAD, DD), jnp.float32),
    )(parts, parts, deg, h, wl, blp, wr)


def kernel(x, edge_index, Wl1, bl1, Wr1, Wl2, bl2, Wr2, Wl3, bl3, Wr3):
    src = edge_index[0].reshape(NW, EPW)
    dst = edge_index[1].reshape(NW, NCH, CH)
    dst_flat = edge_index[1].reshape(NW, EPW)
    xp = jnp.pad(x, ((0, NPAD - NN), (0, 0)))
    zeros_d = jnp.zeros((16, DD), jnp.float32)
    zeros_1 = jnp.zeros((NPAD,), jnp.float32)

    sc_agg, sc_deg = _sc_kernels()
    deg = sc_deg(dst_flat, zeros_1)
    h = xp
    for wl, bl, wr, relu in ((Wl1, bl1, Wr1, True),
                             (Wl2, bl2, Wr2, False),
                             (Wl3, bl3, Wr3, False)):
        parts = sc_agg(h, src, dst, zeros_d)
        h = _dense(parts, deg, h, wl, bl, wr, relu)
    return h[:NN]


# speedup vs baseline: 12.9087x; 1.0058x over previous
import functools

import jax
import jax.numpy as jnp
from jax import lax
from jax.experimental import pallas as pl
from jax.experimental.pallas import tpu as pltpu
from jax.experimental.pallas import tpu_sc as plsc

NN = 10000
EE = 320000
DD = 128
NPAD = 10240
NC = 2
NS = 16
NW = NC * NS
EPW = EE // NW
CH = 80
NCH = EPW // CH
RPT = NPAD // NS


def _mesh():
    return plsc.VectorSubcoreMesh(core_axis_name="c", subcore_axis_name="s",
                                  num_cores=NC, num_subcores=NS)


_NBUF = 3
assert (NCH - 2) % _NBUF == 0


def _sc_agg_body(h_hbm, src_hbm, dst_hbm, zeros_hbm, out_hbm,
                 src_v, dstr_v, stage_v, zero_v, acc_sh,
                 g0, g1, g2, d0, d1, d2, s0, s1, s2):
    gsems = (g0, g1, g2)
    dsems = (d0, d1, d2)
    ssems = (s0, s1, s2)
    c = lax.axis_index("c")
    s = lax.axis_index("s")
    wid = c * NS + s
    base = s * RPT

    def start_fetch(j, b):
        pltpu.async_copy(dst_hbm.at[wid, pl.ds(j, 1)],
                         dstr_v.at[pl.ds(b, 1)], dsems[b])
        pltpu.async_copy(h_hbm.at[src_v.at[pl.ds(j * CH, CH)]],
                         stage_v.at[b], gsems[b])

    def wait_fetch(j, b):
        pltpu.make_async_copy(dst_hbm.at[wid, pl.ds(j, 1)],
                              dstr_v.at[pl.ds(b, 1)], dsems[b]).wait()
        pltpu.make_async_copy(h_hbm.at[src_v.at[pl.ds(j * CH, CH)]],
                              stage_v.at[b], gsems[b]).wait()

    def start_scatter(b):
        pltpu.async_copy(stage_v.at[b], acc_sh.at[dstr_v.at[b]], ssems[b],
                         add=True)

    def wait_scatter(b):
        pltpu.make_async_copy(stage_v.at[b], acc_sh.at[dstr_v.at[b]],
                              ssems[b]).wait()

    pltpu.sync_copy(src_hbm.at[wid], src_v)
    start_fetch(0, 0)
    start_fetch(1, 1)
    pltpu.sync_copy(zeros_hbm, zero_v)
    for k in range(RPT // 16):
        pltpu.sync_copy(zero_v, acc_sh.at[pl.ds(base + k * 16, 16)])
    plsc.subcore_barrier()

    def body(g, carry):
        for b in range(_NBUF):
            j = 3 * g + b
            bn = (b + 2) % _NBUF

            @pl.when(j >= 1)
            def _():
                wait_scatter(bn)
            start_fetch(j + 2, bn)
            wait_fetch(j, b)
            start_scatter(b)
        return carry

    ngrp = (NCH - 2) // _NBUF
    lax.fori_loop(0, ngrp, body, 0)
    for j in range(_NBUF * ngrp, NCH):
        b = j % _NBUF
        wait_scatter((b + 2) % _NBUF)
        wait_fetch(j, b)
        start_scatter(b)
    wait_scatter((NCH - 1) % _NBUF)
    plsc.subcore_barrier()
    pltpu.sync_copy(acc_sh.at[pl.ds(base, RPT)],
                    out_hbm.at[pl.ds(c * NPAD + base, RPT)])


def _sc_deg_body(dst_hbm, zeros_hbm, out_hbm, dst_v, acc_v):
    c = lax.axis_index("c")
    s = lax.axis_index("s")
    wid = c * NS + s
    pltpu.sync_copy(zeros_hbm, acc_v)
    pltpu.sync_copy(dst_hbm.at[wid], dst_v)
    ones = jnp.full((16,), 1.0, jnp.float32)

    def body(i, carry):
        idx = dst_v[pl.ds(i * 16, 16)]
        plsc.addupdate_scatter(acc_v, [idx], ones)
        return carry

    lax.fori_loop(0, EPW // 16, body, 0)
    pltpu.sync_copy(acc_v, out_hbm.at[wid])


@functools.lru_cache(maxsize=None)
def _sc_kernels():
    agg = pl.kernel(
        _sc_agg_body,
        mesh=_mesh(),
        out_type=jax.ShapeDtypeStruct((NC * NPAD, DD), jnp.float32),
        scratch_types=(
            [
                pltpu.VMEM((EPW,), jnp.int32),
                pltpu.VMEM((_NBUF, CH), jnp.int32),
                pltpu.VMEM((_NBUF, CH, DD), jnp.float32),
                pltpu.VMEM((16, DD), jnp.float32),
                pltpu.VMEM_SHARED((NPAD, DD), jnp.float32),
            ]
            + [pltpu.SemaphoreType.DMA] * 9
        ),
    )
    deg = pl.kernel(
        _sc_deg_body,
        mesh=_mesh(),
        compiler_params=pltpu.CompilerParams(needs_layout_passes=False),
        out_type=jax.ShapeDtypeStruct((NW, NPAD), jnp.float32),
        scratch_types=[
            pltpu.VMEM((EPW,), jnp.int32),
            pltpu.VMEM((NPAD,), jnp.float32),
        ],
    )
    return agg, deg


def _tc_body(relu):
    def f(p0, p1, d, h, wl, blp, wr, o):
        deg = jnp.maximum(jnp.sum(d[...], axis=0), 1.0)[:, None]
        agg = (p0[...] + p1[...]) / deg
        y = lax.dot_general(agg, wl[...], (((1,), (1,)), ((), ())),
                            preferred_element_type=jnp.float32)
        y = y + blp[...][0:1, :]
        y = y + lax.dot_general(h[...], wr[...], (((1,), (1,)), ((), ())),
                                preferred_element_type=jnp.float32)
        if relu:
            y = jnp.maximum(y, 0.0)
        o[...] = y
    return f


_BM = 1024
_NBLK = NPAD // _BM


def _dense(parts, deg, h, wl, bl, wr, relu):
    blp = jnp.broadcast_to(bl[None, :], (8, DD))
    return pl.pallas_call(
        _tc_body(relu),
        grid=(_NBLK,),
        in_specs=[
            pl.BlockSpec((_BM, DD), lambda i: (i, 0)),
            pl.BlockSpec((_BM, DD), lambda i: (_NBLK + i, 0)),
            pl.BlockSpec((NW, _BM), lambda i: (0, i)),
            pl.BlockSpec((_BM, DD), lambda i: (i, 0)),
            pl.BlockSpec((DD, DD), lambda i: (0, 0)),
            pl.BlockSpec((8, DD), lambda i: (0, 0)),
            pl.BlockSpec((DD, DD), lambda i: (0, 0)),
        ],
        out_specs=pl.BlockSpec((_BM, DD), lambda i: (i, 0)),
        out_shape=jax.ShapeDtypeStruct((NPAD, DD), jnp.float32),
    )(parts, parts, deg, h, wl, blp, wr)


def kernel(x, edge_index, Wl1, bl1, Wr1, Wl2, bl2, Wr2, Wl3, bl3, Wr3):
    src = edge_index[0].reshape(NW, EPW)
    dst = edge_index[1].reshape(NW, NCH, CH)
    dst_flat = edge_index[1].reshape(NW, EPW)
    xp = jnp.pad(x, ((0, NPAD - NN), (0, 0)))
    zeros_d = jnp.zeros((16, DD), jnp.float32)
    zeros_1 = jnp.zeros((NPAD,), jnp.float32)

    sc_agg, sc_deg = _sc_kernels()
    deg = sc_deg(dst_flat, zeros_1)
    h = xp
    for wl, bl, wr, relu in ((Wl1, bl1, Wr1, True),
                             (Wl2, bl2, Wr2, False),
                             (Wl3, bl3, Wr3, False)):
        parts = sc_agg(h, src, dst, zeros_d)
        h = _dense(parts, deg, h, wl, bl, wr, relu)
    return h[:NN]


# speedup vs baseline: 13.1102x; 1.0156x over previous
import functools

import jax
import jax.numpy as jnp
from jax import lax
from jax.experimental import pallas as pl
from jax.experimental.pallas import tpu as pltpu
from jax.experimental.pallas import tpu_sc as plsc

NN = 10000
EE = 320000
DD = 128
NPAD = 10240
NC = 2
NS = 16
NW = NC * NS
EPW = EE // NW
CH = 80
NCH = EPW // CH
RPT = NPAD // NS


def _mesh():
    return plsc.VectorSubcoreMesh(core_axis_name="c", subcore_axis_name="s",
                                  num_cores=NC, num_subcores=NS)


_NBUF = 3
assert (NCH - 2) % _NBUF == 0


def _sc_agg_body(h_hbm, src_hbm, dst_hbm, zeros_hbm, out_hbm,
                 src_v, dstr_v, stage_v, zero_v, acc_sh,
                 g0, g1, g2, d0, d1, d2, s0, s1, s2, zsem):
    gsems = (g0, g1, g2)
    dsems = (d0, d1, d2)
    ssems = (s0, s1, s2)
    c = lax.axis_index("c")
    s = lax.axis_index("s")
    wid = c * NS + s
    base = s * RPT

    def start_fetch(j, b):
        pltpu.async_copy(dst_hbm.at[wid, pl.ds(j, 1)],
                         dstr_v.at[pl.ds(b, 1)], dsems[b])
        pltpu.async_copy(h_hbm.at[src_v.at[pl.ds(j * CH, CH)]],
                         stage_v.at[b], gsems[b])

    def wait_fetch(j, b):
        pltpu.make_async_copy(dst_hbm.at[wid, pl.ds(j, 1)],
                              dstr_v.at[pl.ds(b, 1)], dsems[b]).wait()
        pltpu.make_async_copy(h_hbm.at[src_v.at[pl.ds(j * CH, CH)]],
                              stage_v.at[b], gsems[b]).wait()

    def start_scatter(b):
        pltpu.async_copy(stage_v.at[b], acc_sh.at[dstr_v.at[b]], ssems[b],
                         add=True)

    def wait_scatter(b):
        pltpu.make_async_copy(stage_v.at[b], acc_sh.at[dstr_v.at[b]],
                              ssems[b]).wait()

    pltpu.sync_copy(src_hbm.at[wid], src_v)
    start_fetch(0, 0)
    start_fetch(1, 1)
    pltpu.sync_copy(zeros_hbm, zero_v)
    nz = RPT // 32
    for k in range(nz):
        pltpu.async_copy(zero_v, acc_sh.at[pl.ds(base + k * 32, 32)], zsem)
    for k in range(nz):
        pltpu.make_async_copy(zero_v, acc_sh.at[pl.ds(base + k * 32, 32)],
                              zsem).wait()
    plsc.subcore_barrier()

    def body(g, carry):
        for b in range(_NBUF):
            j = 3 * g + b
            bn = (b + 2) % _NBUF

            @pl.when(j >= 1)
            def _():
                wait_scatter(bn)
            start_fetch(j + 2, bn)
            wait_fetch(j, b)
            start_scatter(b)
        return carry

    ngrp = (NCH - 2) // _NBUF
    lax.fori_loop(0, ngrp, body, 0)
    for j in range(_NBUF * ngrp, NCH):
        b = j % _NBUF
        wait_scatter((b + 2) % _NBUF)
        wait_fetch(j, b)
        start_scatter(b)
    wait_scatter((NCH - 1) % _NBUF)
    plsc.subcore_barrier()
    pltpu.sync_copy(acc_sh.at[pl.ds(base, RPT)],
                    out_hbm.at[pl.ds(c * NPAD + base, RPT)])


def _sc_deg_body(dst_hbm, zeros_hbm, out_hbm, dst_v, acc_v):
    c = lax.axis_index("c")
    s = lax.axis_index("s")
    wid = c * NS + s
    pltpu.sync_copy(zeros_hbm, acc_v)
    pltpu.sync_copy(dst_hbm.at[wid], dst_v)
    ones = jnp.full((16,), 1.0, jnp.float32)

    def body(i, carry):
        idx = dst_v[pl.ds(i * 16, 16)]
        plsc.addupdate_scatter(acc_v, [idx], ones)
        return carry

    lax.fori_loop(0, EPW // 16, body, 0)
    pltpu.sync_copy(acc_v, out_hbm.at[wid])


@functools.lru_cache(maxsize=None)
def _sc_kernels():
    agg = pl.kernel(
        _sc_agg_body,
        mesh=_mesh(),
        out_type=jax.ShapeDtypeStruct((NC * NPAD, DD), jnp.float32),
        scratch_types=(
            [
                pltpu.VMEM((EPW,), jnp.int32),
                pltpu.VMEM((_NBUF, CH), jnp.int32),
                pltpu.VMEM((_NBUF, CH, DD), jnp.float32),
                pltpu.VMEM((32, DD), jnp.float32),
                pltpu.VMEM_SHARED((NPAD, DD), jnp.float32),
            ]
            + [pltpu.SemaphoreType.DMA] * 10
        ),
    )
    deg = pl.kernel(
        _sc_deg_body,
        mesh=_mesh(),
        compiler_params=pltpu.CompilerParams(needs_layout_passes=False),
        out_type=jax.ShapeDtypeStruct((NW, NPAD), jnp.float32),
        scratch_types=[
            pltpu.VMEM((EPW,), jnp.int32),
            pltpu.VMEM((NPAD,), jnp.float32),
        ],
    )
    return agg, deg


def _tc_body(relu):
    def f(p0, p1, d, h, wl, blp, wr, o):
        deg = jnp.maximum(jnp.sum(d[...], axis=0), 1.0)[:, None]
        agg = (p0[...] + p1[...]) / deg
        y = lax.dot_general(agg, wl[...], (((1,), (1,)), ((), ())),
                            preferred_element_type=jnp.float32)
        y = y + blp[...][0:1, :]
        y = y + lax.dot_general(h[...], wr[...], (((1,), (1,)), ((), ())),
                                preferred_element_type=jnp.float32)
        if relu:
            y = jnp.maximum(y, 0.0)
        o[...] = y
    return f


_BM = 1024
_NBLK = NPAD // _BM


def _dense(parts, deg, h, wl, bl, wr, relu):
    blp = jnp.broadcast_to(bl[None, :], (8, DD))
    return pl.pallas_call(
        _tc_body(relu),
        grid=(_NBLK,),
        in_specs=[
            pl.BlockSpec((_BM, DD), lambda i: (i, 0)),
            pl.BlockSpec((_BM, DD), lambda i: (_NBLK + i, 0)),
            pl.BlockSpec((NW, _BM), lambda i: (0, i)),
            pl.BlockSpec((_BM, DD), lambda i: (i, 0)),
            pl.BlockSpec((DD, DD), lambda i: (0, 0)),
            pl.BlockSpec((8, DD), lambda i: (0, 0)),
            pl.BlockSpec((DD, DD), lambda i: (0, 0)),
        ],
        out_specs=pl.BlockSpec((_BM, DD), lambda i: (i, 0)),
        out_shape=jax.ShapeDtypeStruct((NPAD, DD), jnp.float32),
    )(parts, parts, deg, h, wl, blp, wr)


def kernel(x, edge_index, Wl1, bl1, Wr1, Wl2, bl2, Wr2, Wl3, bl3, Wr3):
    src = edge_index[0].reshape(NW, EPW)
    dst = edge_index[1].reshape(NW, NCH, CH)
    dst_flat = edge_index[1].reshape(NW, EPW)
    xp = jnp.pad(x, ((0, NPAD - NN), (0, 0)))
    zeros_d = jnp.zeros((32, DD), jnp.float32)
    zeros_1 = jnp.zeros((NPAD,), jnp.float32)

    sc_agg, sc_deg = _sc_kernels()
    deg = sc_deg(dst_flat, zeros_1)
    h = xp
    for wl, bl, wr, relu in ((Wl1, bl1, Wr1, True),
                             (Wl2, bl2, Wr2, False),
                             (Wl3, bl3, Wr3, False)):
        parts = sc_agg(h, src, dst, zeros_d)
        h = _dense(parts, deg, h, wl, bl, wr, relu)
    return h[:NN]
